# baseline (speedup 1.0000x reference)
import functools

import jax
import jax.numpy as jnp
from jax import lax
from jax.experimental import pallas as pl
from jax.experimental.pallas import tpu as pltpu
from jax.experimental.pallas import tpu_sc as plsc

N = 10000
E = 320000
D = 128
ALPHA = 0.2

NC = 2
NS = 16
NW = NC * NS
EP = E // NW
CK = 64
NCH = 157
EPP = NCH * CK
PADE = NW * EPP - E
NROW = 10064
NPAD = 10240
RP = 624
BN = 1000


def _dense_body(f_ref, w_ref, alr_ref, ft_ref, a1_ref, a2_ref):
    ft = lax.dot_general(f_ref[...], w_ref[...], (((1,), (1,)), ((), ())),
                         preferred_element_type=jnp.float32)
    ft_ref[...] = ft
    a12 = jnp.dot(ft, alr_ref[...], preferred_element_type=jnp.float32)
    a1_ref[...] = a12[:, 0:1]
    a2_ref[...] = a12[:, 1:2]


_dense = pl.pallas_call(
    _dense_body,
    grid=(N // BN,),
    in_specs=[
        pl.BlockSpec((BN, D), lambda i: (i, 0)),
        pl.BlockSpec((D, D), lambda i: (0, 0)),
        pl.BlockSpec((D, 2), lambda i: (0, 0)),
    ],
    out_specs=[
        pl.BlockSpec((BN, D), lambda i: (i, 0)),
        pl.BlockSpec((BN, 1), lambda i: (i, 0)),
        pl.BlockSpec((BN, 1), lambda i: (i, 0)),
    ],
    out_shape=[
        jax.ShapeDtypeStruct((N, D), jnp.float32),
        jax.ShapeDtypeStruct((N, 1), jnp.float32),
        jax.ShapeDtypeStruct((N, 1), jnp.float32),
    ],
)


@functools.cache
def _build_sc_edge():
    mesh = plsc.VectorSubcoreMesh(
        core_axis_name="c", subcore_axis_name="s", num_cores=NC, num_subcores=NS
    )

    @functools.partial(
        pl.kernel,
        out_type=[
            jax.ShapeDtypeStruct((NC, N, D), jnp.float32),
            jax.ShapeDtypeStruct((NC, NPAD), jnp.float32),
        ],
        mesh=mesh,
        compiler_params=pltpu.CompilerParams(needs_layout_passes=False),
        scratch_types=[
            pltpu.VMEM((N,), jnp.float32),
            pltpu.VMEM((N,), jnp.float32),
            pltpu.VMEM((4, CK), jnp.int32),
            pltpu.VMEM((4, CK), jnp.int32),
            pltpu.VMEM((4, CK), jnp.float32),
            pltpu.VMEM((3, CK, D), jnp.float32),
            pltpu.VMEM_SHARED((NROW, D), jnp.float32),
            pltpu.VMEM_SHARED((NPAD,), jnp.float32),
            pltpu.SemaphoreType.DMA,
            pltpu.SemaphoreType.DMA,
            pltpu.SemaphoreType.DMA,
            pltpu.SemaphoreType.DMA,
        ],
    )
    def _sc_edge(ft_hbm, a1_hbm, a2_hbm, src_hbm, dst_hbm, usum_hbm, den_hbm,
                 a1_v, a2_v, srcc, dstc, sc_v, rows, usum_sh, den_sh,
                 esem, gsem, dsem, ssem):
        c = lax.axis_index("c")
        s = lax.axis_index("s")
        wid = s * NC + c

        a1d = pltpu.async_copy(a1_hbm, a1_v, esem)
        a2d = pltpu.async_copy(a2_hbm, a2_v, esem)

        z16 = jnp.zeros((16,), jnp.float32)

        def _zrow(i, carry):
            r = rows.at[0, i]
            for j in range(D // 16):
                r[pl.ds(j * 16, 16)] = z16
            return carry

        lax.fori_loop(0, CK, _zrow, 0)

        def _zcp(k, carry):
            pltpu.sync_copy(rows.at[0], usum_sh.at[pl.ds(s * RP + k * CK, CK)])
            return carry

        lax.fori_loop(0, 10, _zcp, 0)

        @pl.when(s == NS - 1)
        def _():
            pltpu.sync_copy(rows.at[0, pl.ds(0, NROW - N)],
                            usum_sh.at[pl.ds(N, NROW - N)])

        def _zden(k, carry):
            pltpu.sync_copy(rows.at[0, 0],
                            den_sh.at[pl.ds(s * 640 + k * 128, 128)])
            return carry

        lax.fori_loop(0, 5, _zden, 0)
        a1d.wait()
        a2d.wait()
        plsc.subcore_barrier()

        def _compute_s(ch, m):
            for j in range(CK // 16):
                srcv = srcc.at[m][pl.ds(j * 16, 16)]
                dstv = dstc.at[m][pl.ds(j * 16, 16)]
                dstv = jnp.minimum(dstv, N - 1)
                e = (plsc.load_gather(a1_v, [srcv])
                     + plsc.load_gather(a2_v, [dstv]))
                e = jnp.where(e > 0, e, ALPHA * e)
                sc_v.at[m][pl.ds(j * 16, 16)] = jnp.exp(e)

        def _issue_idx(ch):
            m = lax.rem(ch, 4)
            pltpu.async_copy(src_hbm.at[wid, ch], srcc.at[m], esem)
            pltpu.async_copy(dst_hbm.at[wid, ch], dstc.at[m], esem)

        def _wait_idx(ch):
            m = lax.rem(ch, 4)
            pltpu.make_async_copy(src_hbm.at[wid, ch], srcc.at[m], esem).wait()
            pltpu.make_async_copy(dst_hbm.at[wid, ch], dstc.at[m], esem).wait()

        def _issue_gather(ch):
            m = lax.rem(ch, 4)
            p = lax.rem(ch, 3)
            pltpu.async_copy(ft_hbm.at[srcc.at[m]], rows.at[p], gsem)

        def _wait_gather(ch):
            m = lax.rem(ch, 4)
            p = lax.rem(ch, 3)
            pltpu.make_async_copy(ft_hbm.at[srcc.at[m]], rows.at[p],
                                  gsem).wait()

        def _issue_den(ch):
            m = lax.rem(ch, 4)
            pltpu.async_copy(sc_v.at[m], den_sh.at[dstc.at[m]], dsem, add=True)

        def _wait_den(ch):
            m = lax.rem(ch, 4)
            pltpu.make_async_copy(sc_v.at[m], den_sh.at[dstc.at[m]],
                                  dsem).wait()

        def _issue_scat(ch):
            m = lax.rem(ch, 4)
            p = lax.rem(ch, 3)
            pltpu.async_copy(rows.at[p], usum_sh.at[dstc.at[m]], ssem,
                             add=True)

        def _wait_scat(ch):
            m = lax.rem(ch, 4)
            p = lax.rem(ch, 3)
            pltpu.make_async_copy(rows.at[p], usum_sh.at[dstc.at[m]],
                                  ssem).wait()

        _issue_idx(0)
        _wait_idx(0)
        _issue_gather(0)
        _compute_s(0, 0)
        _issue_den(0)
        _issue_idx(1)

        def _iter(ch, carry):
            @pl.when(ch < NCH - 1)
            def _():
                _wait_idx(ch + 1)
                _compute_s(ch + 1, lax.rem(ch + 1, 4))

            @pl.when(ch >= 2)
            def _():
                _wait_scat(ch - 2)

            @pl.when(ch >= 1)
            def _():
                _wait_den(ch - 1)

            @pl.when(ch < NCH - 1)
            def _():
                _issue_gather(ch + 1)
                _issue_den(ch + 1)

                @pl.when(ch < NCH - 2)
                def _():
                    _issue_idx(ch + 2)

            _wait_gather(ch)

            m0 = lax.rem(ch, 4)
            p0 = lax.rem(ch, 3)

            def _splat(i):
                return plsc.load_gather(
                    sc_v, [jnp.full((16,), m0, jnp.int32),
                           jnp.full((16,), i, jnp.int32)])

            def _scale(i, sp):
                sp_next = _splat(jnp.minimum(i + 1, CK - 1))
                r = rows.at[p0, i]
                for j in range(D // 16):
                    r[pl.ds(j * 16, 16)] = r[pl.ds(j * 16, 16)] * sp
                return sp_next

            lax.fori_loop(0, CK, _scale, _splat(jnp.int32(0)))
            _issue_scat(ch)
            return carry

        lax.fori_loop(0, NCH, _iter, 0)
        _wait_den(NCH - 1)
        _wait_scat(NCH - 2)
        _wait_scat(NCH - 1)
        plsc.subcore_barrier()

        @pl.when(s < NS - 1)
        def _():
            pltpu.sync_copy(usum_sh.at[pl.ds(s * RP, RP)],
                            usum_hbm.at[c, pl.ds(s * RP, RP)])

        @pl.when(s == NS - 1)
        def _():
            pltpu.sync_copy(usum_sh.at[pl.ds((NS - 1) * RP, N - (NS - 1) * RP)],
                            usum_hbm.at[c, pl.ds((NS - 1) * RP, N - (NS - 1) * RP)])

        pltpu.sync_copy(den_sh.at[pl.ds(s * 640, 640)],
                        den_hbm.at[c, pl.ds(s * 640, 640)])

    return _sc_edge


def _norm_body(u_ref, d_ref, o_ref):
    u = u_ref[0] + u_ref[1]
    den = d_ref[0] + d_ref[1]
    safe = jnp.where(den > 0, den, 1.0)
    o_ref[...] = u / safe


_norm = pl.pallas_call(
    _norm_body,
    grid=(N // BN,),
    in_specs=[
        pl.BlockSpec((2, BN, D), lambda i: (0, i, 0)),
        pl.BlockSpec((2, BN, 1), lambda i: (0, i, 0)),
    ],
    out_specs=pl.BlockSpec((BN, D), lambda i: (i, 0)),
    out_shape=jax.ShapeDtypeStruct((N, D), jnp.float32),
)


def kernel(feature, edge_index, W, attn_l, attn_r):
    alr = jnp.stack([attn_l.reshape(D), attn_r.reshape(D)], axis=1)
    ft, a1o, a2o = _dense(feature, W, alr)
    a1 = a1o.reshape(N)
    a2 = a2o.reshape(N)
    src3 = jnp.concatenate(
        [edge_index[0], jnp.zeros((PADE,), jnp.int32)]).reshape(NW, NCH, CK)
    padd = N + (jnp.arange(PADE, dtype=jnp.int32) % 64)
    dst3 = jnp.concatenate([edge_index[1], padd]).reshape(NW, NCH, CK)
    usum, den = _build_sc_edge()(ft, a1, a2, src3, dst3)
    den3 = den[:, :N].reshape(NC, N, 1)
    out = _norm(usum, den3)
    return out.reshape(N, 1, D)


# speedup vs baseline: 2.5039x; 2.5039x over previous
import functools

import jax
import jax.numpy as jnp
from jax import lax
from jax.experimental import pallas as pl
from jax.experimental.pallas import tpu as pltpu
from jax.experimental.pallas import tpu_sc as plsc

N = 10000
E = 320000
D = 128
ALPHA = 0.2

NC = 2
NS = 16
NW = NC * NS
EP = E // NW
CK = 80
NCH = EP // CK
NPAD = 10240
RP = 624
BN = 1000


def _dense_body(f_ref, w_ref, alr_ref, ft_ref, a1_ref, a2_ref):
    ft = lax.dot_general(f_ref[...], w_ref[...], (((1,), (1,)), ((), ())),
                         preferred_element_type=jnp.float32)
    ft_ref[...] = ft
    a12 = jnp.dot(ft, alr_ref[...], preferred_element_type=jnp.float32)
    a1_ref[...] = a12[:, 0:1]
    a2_ref[...] = a12[:, 1:2]


_dense = pl.pallas_call(
    _dense_body,
    grid=(N // BN,),
    in_specs=[
        pl.BlockSpec((BN, D), lambda i: (i, 0)),
        pl.BlockSpec((D, D), lambda i: (0, 0)),
        pl.BlockSpec((D, 2), lambda i: (0, 0)),
    ],
    out_specs=[
        pl.BlockSpec((BN, D), lambda i: (i, 0)),
        pl.BlockSpec((BN, 1), lambda i: (i, 0)),
        pl.BlockSpec((BN, 1), lambda i: (i, 0)),
    ],
    out_shape=[
        jax.ShapeDtypeStruct((N, D), jnp.float32),
        jax.ShapeDtypeStruct((N, 1), jnp.float32),
        jax.ShapeDtypeStruct((N, 1), jnp.float32),
    ],
)


@functools.cache
def _build_sc_edge():
    mesh = plsc.VectorSubcoreMesh(
        core_axis_name="c", subcore_axis_name="s", num_cores=NC, num_subcores=NS
    )

    @functools.partial(
        pl.kernel,
        out_type=[
            jax.ShapeDtypeStruct((NC, N, D), jnp.float32),
            jax.ShapeDtypeStruct((NC, NPAD), jnp.float32),
        ],
        mesh=mesh,
        compiler_params=pltpu.CompilerParams(needs_layout_passes=False),
        scratch_types=[
            pltpu.VMEM((N,), jnp.float32),
            pltpu.VMEM((N,), jnp.float32),
            pltpu.VMEM((4, CK), jnp.int32),
            pltpu.VMEM((4, CK), jnp.int32),
            pltpu.VMEM((4, CK), jnp.float32),
            pltpu.VMEM((2, CK, D), jnp.float32),
            pltpu.VMEM_SHARED((N, D), jnp.float32),
            pltpu.VMEM_SHARED((NPAD,), jnp.float32),
            pltpu.SemaphoreType.DMA,
            pltpu.SemaphoreType.DMA,
            pltpu.SemaphoreType.DMA,
            pltpu.SemaphoreType.DMA,
        ],
    )
    def _sc_edge(ft_hbm, a1_hbm, a2_hbm, src_hbm, dst_hbm, usum_hbm, den_hbm,
                 a1_v, a2_v, srcc, dstc, sc_v, rows, usum_sh, den_sh,
                 esem, gsem, dsem, ssem):
        c = lax.axis_index("c")
        s = lax.axis_index("s")
        wid = s * NC + c

        a1d = pltpu.async_copy(a1_hbm, a1_v, esem)
        a2d = pltpu.async_copy(a2_hbm, a2_v, esem)

        z16 = jnp.zeros((16,), jnp.float32)

        def _zrow(i, carry):
            r = rows.at[0, i]
            for j in range(D // 16):
                r[pl.ds(j * 16, 16)] = z16
            return carry

        lax.fori_loop(0, CK, _zrow, 0)

        def _zcp(k, carry):
            pltpu.sync_copy(rows.at[0], usum_sh.at[pl.ds(s * RP + k * CK, CK)])
            return carry

        lax.fori_loop(0, 7, _zcp, 0)
        pltpu.sync_copy(rows.at[0, pl.ds(0, 64)],
                        usum_sh.at[pl.ds(s * RP + 560, 64)])

        @pl.when(s == NS - 1)
        def _():
            pltpu.sync_copy(rows.at[0], usum_sh.at[pl.ds(N - CK, CK)])

        def _zden(k, carry):
            pltpu.sync_copy(rows.at[0, 0],
                            den_sh.at[pl.ds(s * 640 + k * 128, 128)])
            return carry

        lax.fori_loop(0, 5, _zden, 0)
        a1d.wait()
        a2d.wait()
        plsc.subcore_barrier()

        def _compute_s(ch, m):
            for j in range(CK // 16):
                srcv = srcc.at[m][pl.ds(j * 16, 16)]
                dstv = dstc.at[m][pl.ds(j * 16, 16)]
                e = (plsc.load_gather(a1_v, [srcv])
                     + plsc.load_gather(a2_v, [dstv]))
                e = jnp.where(e > 0, e, ALPHA * e)
                sc_v.at[m][pl.ds(j * 16, 16)] = jnp.exp(e)

        def _issue_idx(ch):
            m = lax.rem(ch, 4)
            pltpu.async_copy(src_hbm.at[wid, ch], srcc.at[m], esem)
            pltpu.async_copy(dst_hbm.at[wid, ch], dstc.at[m], esem)

        def _wait_idx(ch):
            m = lax.rem(ch, 4)
            pltpu.make_async_copy(src_hbm.at[wid, ch], srcc.at[m], esem).wait()
            pltpu.make_async_copy(dst_hbm.at[wid, ch], dstc.at[m], esem).wait()

        def _issue_gather(ch):
            m = lax.rem(ch, 4)
            p = lax.rem(ch, 2)
            pltpu.async_copy(ft_hbm.at[srcc.at[m]], rows.at[p], gsem)

        def _wait_gather(ch):
            m = lax.rem(ch, 4)
            p = lax.rem(ch, 2)
            pltpu.make_async_copy(ft_hbm.at[srcc.at[m]], rows.at[p],
                                  gsem).wait()

        def _issue_den(ch):
            m = lax.rem(ch, 4)
            pltpu.async_copy(sc_v.at[m], den_sh.at[dstc.at[m]], dsem, add=True)

        def _wait_den(ch):
            m = lax.rem(ch, 4)
            pltpu.make_async_copy(sc_v.at[m], den_sh.at[dstc.at[m]],
                                  dsem).wait()

        def _issue_scat(ch):
            m = lax.rem(ch, 4)
            p = lax.rem(ch, 2)
            pltpu.async_copy(rows.at[p], usum_sh.at[dstc.at[m]], ssem,
                             add=True)

        def _wait_scat(ch):
            m = lax.rem(ch, 4)
            p = lax.rem(ch, 2)
            pltpu.make_async_copy(rows.at[p], usum_sh.at[dstc.at[m]],
                                  ssem).wait()

        _issue_idx(0)
        _wait_idx(0)
        _issue_gather(0)
        _compute_s(0, 0)
        _issue_den(0)
        _issue_idx(1)

        def _iter(ch, carry):
            @pl.when(ch < NCH - 1)
            def _():
                _wait_idx(ch + 1)
                _compute_s(ch + 1, lax.rem(ch + 1, 4))

            @pl.when(ch >= 1)
            def _():
                _wait_scat(ch - 1)
                _wait_den(ch - 1)

            @pl.when(ch < NCH - 1)
            def _():
                _issue_gather(ch + 1)
                _issue_den(ch + 1)

                @pl.when(ch < NCH - 2)
                def _():
                    _issue_idx(ch + 2)

            _wait_gather(ch)

            m0 = lax.rem(ch, 4)
            p0 = lax.rem(ch, 2)

            def _splat(i):
                return plsc.load_gather(
                    sc_v, [jnp.full((16,), m0, jnp.int32),
                           jnp.full((16,), i, jnp.int32)])

            def _scale(i, carry2):
                sp0, sp1 = carry2
                spn0 = _splat(jnp.minimum(2 * i + 2, CK - 1))
                spn1 = _splat(jnp.minimum(2 * i + 3, CK - 1))
                r0 = rows.at[p0, 2 * i]
                r1 = rows.at[p0, 2 * i + 1]
                for j in range(D // 16):
                    r0[pl.ds(j * 16, 16)] = r0[pl.ds(j * 16, 16)] * sp0
                for j in range(D // 16):
                    r1[pl.ds(j * 16, 16)] = r1[pl.ds(j * 16, 16)] * sp1
                return (spn0, spn1)

            lax.fori_loop(0, CK // 2, _scale,
                          (_splat(jnp.int32(0)), _splat(jnp.int32(1))))
            _issue_scat(ch)
            return carry

        lax.fori_loop(0, NCH, _iter, 0)
        _wait_den(NCH - 1)
        _wait_scat(NCH - 1)
        plsc.subcore_barrier()

        @pl.when(s < NS - 1)
        def _():
            pltpu.sync_copy(usum_sh.at[pl.ds(s * RP, RP)],
                            usum_hbm.at[c, pl.ds(s * RP, RP)])

        @pl.when(s == NS - 1)
        def _():
            pltpu.sync_copy(usum_sh.at[pl.ds((NS - 1) * RP, N - (NS - 1) * RP)],
                            usum_hbm.at[c, pl.ds((NS - 1) * RP, N - (NS - 1) * RP)])

        pltpu.sync_copy(den_sh.at[pl.ds(s * 640, 640)],
                        den_hbm.at[c, pl.ds(s * 640, 640)])

    return _sc_edge


def _norm_body(u_ref, d_ref, o_ref):
    u = u_ref[0] + u_ref[1]
    den = d_ref[0] + d_ref[1]
    safe = jnp.where(den > 0, den, 1.0)
    o_ref[...] = u / safe


_norm = pl.pallas_call(
    _norm_body,
    grid=(N // BN,),
    in_specs=[
        pl.BlockSpec((2, BN, D), lambda i: (0, i, 0)),
        pl.BlockSpec((2, BN, 1), lambda i: (0, i, 0)),
    ],
    out_specs=pl.BlockSpec((BN, D), lambda i: (i, 0)),
    out_shape=jax.ShapeDtypeStruct((N, D), jnp.float32),
)


def kernel(feature, edge_index, W, attn_l, attn_r):
    alr = jnp.stack([attn_l.reshape(D), attn_r.reshape(D)], axis=1)
    ft, a1o, a2o = _dense(feature, W, alr)
    a1 = a1o.reshape(N)
    a2 = a2o.reshape(N)
    src3 = edge_index[0].reshape(NW, NCH, CK)
    dst3 = edge_index[1].reshape(NW, NCH, CK)
    usum, den = _build_sc_edge()(ft, a1, a2, src3, dst3)
    den3 = den[:, :N].reshape(NC, N, 1)
    out = _norm(usum, den3)
    return out.reshape(N, 1, D)


# speedup vs baseline: 2.6273x; 1.0493x over previous
import functools

import jax
import jax.numpy as jnp
from jax import lax
from jax.experimental import pallas as pl
from jax.experimental.pallas import tpu as pltpu
from jax.experimental.pallas import tpu_sc as plsc

N = 10000
E = 320000
D = 128
ALPHA = 0.2

NC = 2
NS = 16
NW = NC * NS
EP = E // NW
CK = 80
NCH = EP // CK
NPAD = 10240
RP = 624
BN = 1000


def _dense_body(f_ref, w_ref, alr_ref, ft_ref, a1_ref, a2_ref):
    ft = lax.dot_general(f_ref[...], w_ref[...], (((1,), (1,)), ((), ())),
                         preferred_element_type=jnp.float32)
    ft_ref[...] = ft
    a12 = jnp.dot(ft, alr_ref[...], preferred_element_type=jnp.float32)
    a1_ref[...] = a12[:, 0:1]
    a2_ref[...] = a12[:, 1:2]


_dense = pl.pallas_call(
    _dense_body,
    grid=(N // BN,),
    in_specs=[
        pl.BlockSpec((BN, D), lambda i: (i, 0)),
        pl.BlockSpec((D, D), lambda i: (0, 0)),
        pl.BlockSpec((D, 2), lambda i: (0, 0)),
    ],
    out_specs=[
        pl.BlockSpec((BN, D), lambda i: (i, 0)),
        pl.BlockSpec((BN, 1), lambda i: (i, 0)),
        pl.BlockSpec((BN, 1), lambda i: (i, 0)),
    ],
    out_shape=[
        jax.ShapeDtypeStruct((N, D), jnp.float32),
        jax.ShapeDtypeStruct((N, 1), jnp.float32),
        jax.ShapeDtypeStruct((N, 1), jnp.float32),
    ],
)


@functools.cache
def _build_sc_edge():
    mesh = plsc.VectorSubcoreMesh(
        core_axis_name="c", subcore_axis_name="s", num_cores=NC, num_subcores=NS
    )

    @functools.partial(
        pl.kernel,
        out_type=[
            jax.ShapeDtypeStruct((NC, N, D), jnp.float32),
            jax.ShapeDtypeStruct((NC, NPAD), jnp.float32),
        ],
        mesh=mesh,
        compiler_params=pltpu.CompilerParams(needs_layout_passes=False),
        scratch_types=[
            pltpu.VMEM((N,), jnp.float32),
            pltpu.VMEM((N,), jnp.float32),
            pltpu.VMEM((4, CK), jnp.int32),
            pltpu.VMEM((4, CK), jnp.int32),
            pltpu.VMEM((4, CK), jnp.float32),
            pltpu.VMEM((2, CK, D), jnp.float32),
            pltpu.VMEM_SHARED((N, D), jnp.float32),
            pltpu.VMEM_SHARED((NPAD,), jnp.float32),
            pltpu.SemaphoreType.DMA,
            pltpu.SemaphoreType.DMA,
            pltpu.SemaphoreType.DMA,
            pltpu.SemaphoreType.DMA,
        ],
    )
    def _sc_edge(ft_hbm, a1_hbm, a2_hbm, ei_hbm, usum_hbm, den_hbm,
                 a1_v, a2_v, srcc, dstc, sc_v, rows, usum_sh, den_sh,
                 esem, gsem, dsem, ssem):
        c = lax.axis_index("c")
        s = lax.axis_index("s")
        wid = s * NC + c

        a1d = pltpu.async_copy(a1_hbm, a1_v, esem)
        a2d = pltpu.async_copy(a2_hbm, a2_v, esem)

        z16 = jnp.zeros((16,), jnp.float32)

        def _zrow(i, carry):
            r = rows.at[0, i]
            for j in range(D // 16):
                r[pl.ds(j * 16, 16)] = z16
            return carry

        lax.fori_loop(0, CK, _zrow, 0)

        def _zcp(k, carry):
            pltpu.sync_copy(rows.at[0], usum_sh.at[pl.ds(s * RP + k * CK, CK)])
            return carry

        lax.fori_loop(0, 7, _zcp, 0)
        pltpu.sync_copy(rows.at[0, pl.ds(0, 64)],
                        usum_sh.at[pl.ds(s * RP + 560, 64)])

        @pl.when(s == NS - 1)
        def _():
            pltpu.sync_copy(rows.at[0], usum_sh.at[pl.ds(N - CK, CK)])

        def _zden(k, carry):
            pltpu.sync_copy(rows.at[0, 0],
                            den_sh.at[pl.ds(s * 640 + k * 128, 128)])
            return carry

        lax.fori_loop(0, 5, _zden, 0)
        a1d.wait()
        a2d.wait()
        plsc.subcore_barrier()

        def _compute_s(ch, m):
            for j in range(CK // 16):
                srcv = srcc.at[m][pl.ds(j * 16, 16)]
                dstv = dstc.at[m][pl.ds(j * 16, 16)]
                e = (plsc.load_gather(a1_v, [srcv])
                     + plsc.load_gather(a2_v, [dstv]))
                e = jnp.where(e > 0, e, ALPHA * e)
                sc_v.at[m][pl.ds(j * 16, 16)] = jnp.exp(e)

        def _issue_idx(ch):
            m = lax.rem(ch, 4)
            pltpu.async_copy(ei_hbm.at[0, wid, ch], srcc.at[m], esem)
            pltpu.async_copy(ei_hbm.at[1, wid, ch], dstc.at[m], esem)

        def _wait_idx(ch):
            m = lax.rem(ch, 4)
            pltpu.make_async_copy(ei_hbm.at[0, wid, ch], srcc.at[m],
                                  esem).wait()
            pltpu.make_async_copy(ei_hbm.at[1, wid, ch], dstc.at[m],
                                  esem).wait()

        def _issue_gather(ch):
            m = lax.rem(ch, 4)
            p = lax.rem(ch, 2)
            pltpu.async_copy(ft_hbm.at[srcc.at[m]], rows.at[p], gsem)

        def _wait_gather(ch):
            m = lax.rem(ch, 4)
            p = lax.rem(ch, 2)
            pltpu.make_async_copy(ft_hbm.at[srcc.at[m]], rows.at[p],
                                  gsem).wait()

        def _issue_den(ch):
            m = lax.rem(ch, 4)
            pltpu.async_copy(sc_v.at[m], den_sh.at[dstc.at[m]], dsem, add=True)

        def _wait_den(ch):
            m = lax.rem(ch, 4)
            pltpu.make_async_copy(sc_v.at[m], den_sh.at[dstc.at[m]],
                                  dsem).wait()

        def _issue_scat(ch):
            m = lax.rem(ch, 4)
            p = lax.rem(ch, 2)
            pltpu.async_copy(rows.at[p], usum_sh.at[dstc.at[m]], ssem,
                             add=True)

        def _wait_scat(ch):
            m = lax.rem(ch, 4)
            p = lax.rem(ch, 2)
            pltpu.make_async_copy(rows.at[p], usum_sh.at[dstc.at[m]],
                                  ssem).wait()

        _issue_idx(0)
        _wait_idx(0)
        _issue_gather(0)
        _compute_s(0, 0)
        _issue_den(0)
        _issue_idx(1)

        def _iter(ch, carry):
            @pl.when(ch < NCH - 1)
            def _():
                _wait_idx(ch + 1)
                _compute_s(ch + 1, lax.rem(ch + 1, 4))

            @pl.when(ch >= 1)
            def _():
                _wait_scat(ch - 1)
                _wait_den(ch - 1)

            @pl.when(ch < NCH - 1)
            def _():
                _issue_gather(ch + 1)
                _issue_den(ch + 1)

                @pl.when(ch < NCH - 2)
                def _():
                    _issue_idx(ch + 2)

            _wait_gather(ch)

            m0 = lax.rem(ch, 4)
            p0 = lax.rem(ch, 2)

            def _splat(i):
                return plsc.load_gather(
                    sc_v, [jnp.full((16,), m0, jnp.int32),
                           jnp.full((16,), i, jnp.int32)])

            def _scale(i, carry2):
                sp0, sp1 = carry2
                spn0 = _splat(jnp.minimum(2 * i + 2, CK - 1))
                spn1 = _splat(jnp.minimum(2 * i + 3, CK - 1))
                r0 = rows.at[p0, 2 * i]
                r1 = rows.at[p0, 2 * i + 1]
                for j in range(D // 16):
                    r0[pl.ds(j * 16, 16)] = r0[pl.ds(j * 16, 16)] * sp0
                for j in range(D // 16):
                    r1[pl.ds(j * 16, 16)] = r1[pl.ds(j * 16, 16)] * sp1
                return (spn0, spn1)

            lax.fori_loop(0, CK // 2, _scale,
                          (_splat(jnp.int32(0)), _splat(jnp.int32(1))))
            _issue_scat(ch)
            return carry

        lax.fori_loop(0, NCH, _iter, 0)
        _wait_den(NCH - 1)
        _wait_scat(NCH - 1)
        plsc.subcore_barrier()

        @pl.when(s < NS - 1)
        def _():
            pltpu.sync_copy(usum_sh.at[pl.ds(s * RP, RP)],
                            usum_hbm.at[c, pl.ds(s * RP, RP)])

        @pl.when(s == NS - 1)
        def _():
            pltpu.sync_copy(usum_sh.at[pl.ds((NS - 1) * RP, N - (NS - 1) * RP)],
                            usum_hbm.at[c, pl.ds((NS - 1) * RP, N - (NS - 1) * RP)])

        pltpu.sync_copy(den_sh.at[pl.ds(s * 640, 640)],
                        den_hbm.at[c, pl.ds(s * 640, 640)])

    return _sc_edge


def _norm_body(u_ref, d_ref, o_ref):
    u = u_ref[0] + u_ref[1]
    den = d_ref[0] + d_ref[1]
    safe = jnp.where(den > 0, den, 1.0)
    o_ref[...] = (u / safe)[:, None, :]


_norm = pl.pallas_call(
    _norm_body,
    grid=(N // BN,),
    in_specs=[
        pl.BlockSpec((2, BN, D), lambda i: (0, i, 0)),
        pl.BlockSpec((2, BN, 1), lambda i: (0, i, 0)),
    ],
    out_specs=pl.BlockSpec((BN, 1, D), lambda i: (i, 0, 0)),
    out_shape=jax.ShapeDtypeStruct((N, 1, D), jnp.float32),
)


def kernel(feature, edge_index, W, attn_l, attn_r):
    alr = jnp.stack([attn_l.reshape(D), attn_r.reshape(D)], axis=1)
    ft, a1o, a2o = _dense(feature, W, alr)
    ei4 = edge_index.reshape(2, NW, NCH, CK)
    usum, den = _build_sc_edge()(ft, a1o.reshape(N), a2o.reshape(N), ei4)
    den3 = den[:, :N].reshape(NC, N, 1)
    return _norm(usum, den3)
